# baseline (device time: 242609 ns/iter reference)
import jax
import jax.numpy as jnp
from jax import lax
from jax.experimental import pallas as pl
from jax.experimental.pallas import tpu as pltpu

N_DEV = 32


def kernel(x, w_mat, scale_x, scale_w):
    m_per, k = x.shape
    _, n_per = w_mat.shape

    x8 = x.astype(jnp.float8_e4m3fn)
    wb = w_mat.astype(jnp.bfloat16)
    sx = scale_x.reshape(1, 1).astype(jnp.float32)
    sw = scale_w.reshape(1, 1).astype(jnp.float32)

    def body(x_ref, w_ref, sx_ref, sw_ref, out_ref, xg_ref, send_sems, recv_sems):
        my = lax.axis_index("i")
        left = lax.rem(my + N_DEV - 1, N_DEV)
        right = lax.rem(my + 1, N_DEV)

        barrier_sem = pltpu.get_barrier_semaphore()
        for nbr in (left, right):
            pl.semaphore_signal(
                barrier_sem, inc=1,
                device_id=(nbr,), device_id_type=pl.DeviceIdType.MESH,
            )
        pl.semaphore_wait(barrier_sem, 2)

        alpha = sx_ref[0, 0] * sw_ref[0, 0]

        def gemm(origin):
            chunk = xg_ref[origin].astype(jnp.bfloat16)
            acc = jnp.dot(chunk, w_ref[:, :], preferred_element_type=jnp.float32)
            out_ref[pl.ds(origin * m_per, m_per), :] = jnp.maximum(acc * alpha, 0.0)

        xg_ref[my] = x_ref[:, :]

        for h in range(N_DEV - 1):
            src_origin = lax.rem(my - h + N_DEV, N_DEV)
            rdma = pltpu.make_async_remote_copy(
                src_ref=xg_ref.at[src_origin],
                dst_ref=xg_ref.at[src_origin],
                send_sem=send_sems.at[h],
                recv_sem=recv_sems.at[h],
                device_id=(right,),
                device_id_type=pl.DeviceIdType.MESH,
            )
            rdma.start()
            gemm(src_origin)
            rdma.wait_send()
            rdma.wait_recv()
        gemm(right)

    out_shape = jax.ShapeDtypeStruct((N_DEV * m_per, n_per), jnp.float32)
    return pl.pallas_call(
        body,
        out_shape=out_shape,
        in_specs=[
            pl.BlockSpec(memory_space=pltpu.VMEM),
            pl.BlockSpec(memory_space=pltpu.VMEM),
            pl.BlockSpec(memory_space=pltpu.SMEM),
            pl.BlockSpec(memory_space=pltpu.SMEM),
        ],
        out_specs=pl.BlockSpec(memory_space=pltpu.VMEM),
        scratch_shapes=[
            pltpu.VMEM((N_DEV, m_per, k), jnp.float8_e4m3fn),
            pltpu.SemaphoreType.DMA((N_DEV - 1,)),
            pltpu.SemaphoreType.DMA((N_DEV - 1,)),
        ],
        compiler_params=pltpu.CompilerParams(collective_id=0),
    )(x8, wb, sx, sw)


# device time: 212732 ns/iter; 1.1404x vs baseline; 1.1404x over previous
import jax
import jax.numpy as jnp
from jax import lax
from jax.experimental import pallas as pl
from jax.experimental.pallas import tpu as pltpu

N_DEV = 32
FWD = N_DEV // 2
BWD = N_DEV - 1 - FWD


def kernel(x, w_mat, scale_x, scale_w):
    m_per, k = x.shape
    _, n_per = w_mat.shape

    x8 = x.astype(jnp.float8_e4m3fn)
    wb = w_mat.astype(jnp.bfloat16)
    sx = scale_x.reshape(1, 1).astype(jnp.float32)
    sw = scale_w.reshape(1, 1).astype(jnp.float32)

    def body(x_ref, w_ref, sx_ref, sw_ref, out_ref, xg_ref,
             fs_sems, fr_sems, bs_sems, br_sems):
        my = lax.axis_index("i")
        left = lax.rem(my + N_DEV - 1, N_DEV)
        right = lax.rem(my + 1, N_DEV)

        barrier_sem = pltpu.get_barrier_semaphore()
        for nbr in (left, right):
            pl.semaphore_signal(
                barrier_sem, inc=1,
                device_id=(nbr,), device_id_type=pl.DeviceIdType.MESH,
            )
        pl.semaphore_wait(barrier_sem, 2)

        alpha = sx_ref[0, 0] * sw_ref[0, 0]

        def gemm(origin):
            chunk = xg_ref[origin].astype(jnp.bfloat16)
            acc = jnp.dot(chunk, w_ref[:, :], preferred_element_type=jnp.float32)
            out_ref[pl.ds(origin * m_per, m_per), :] = jnp.maximum(acc * alpha, 0.0)

        xg_ref[my] = x_ref[:, :]

        for h in range(FWD):
            fwd_origin = lax.rem(my - h + N_DEV, N_DEV)
            fwd = pltpu.make_async_remote_copy(
                src_ref=xg_ref.at[fwd_origin],
                dst_ref=xg_ref.at[fwd_origin],
                send_sem=fs_sems.at[h],
                recv_sem=fr_sems.at[h],
                device_id=(right,),
                device_id_type=pl.DeviceIdType.MESH,
            )
            fwd.start()
            if h < BWD:
                bwd_origin = lax.rem(my + h, N_DEV)
                bwd = pltpu.make_async_remote_copy(
                    src_ref=xg_ref.at[bwd_origin],
                    dst_ref=xg_ref.at[bwd_origin],
                    send_sem=bs_sems.at[h],
                    recv_sem=br_sems.at[h],
                    device_id=(left,),
                    device_id_type=pl.DeviceIdType.MESH,
                )
                bwd.start()
            gemm(fwd_origin)
            if h >= 1:
                gemm(lax.rem(my + h, N_DEV))
            fwd.wait_send()
            fwd.wait_recv()
            if h < BWD:
                bwd.wait_send()
                bwd.wait_recv()
        gemm(lax.rem(my + FWD, N_DEV))

    out_shape = jax.ShapeDtypeStruct((N_DEV * m_per, n_per), jnp.float32)
    return pl.pallas_call(
        body,
        out_shape=out_shape,
        in_specs=[
            pl.BlockSpec(memory_space=pltpu.VMEM),
            pl.BlockSpec(memory_space=pltpu.VMEM),
            pl.BlockSpec(memory_space=pltpu.SMEM),
            pl.BlockSpec(memory_space=pltpu.SMEM),
        ],
        out_specs=pl.BlockSpec(memory_space=pltpu.VMEM),
        scratch_shapes=[
            pltpu.VMEM((N_DEV, m_per, k), jnp.float8_e4m3fn),
            pltpu.SemaphoreType.DMA((FWD,)),
            pltpu.SemaphoreType.DMA((FWD,)),
            pltpu.SemaphoreType.DMA((BWD,)),
            pltpu.SemaphoreType.DMA((BWD,)),
        ],
        compiler_params=pltpu.CompilerParams(collective_id=0),
    )(x8, wb, sx, sw)


# device time: 192504 ns/iter; 1.2603x vs baseline; 1.1051x over previous
import jax
import jax.numpy as jnp
from jax import lax
from jax.experimental import pallas as pl
from jax.experimental.pallas import tpu as pltpu

N_DEV = 32
FWD = N_DEV // 2
BWD = N_DEV - 1 - FWD


def kernel(x, w_mat, scale_x, scale_w):
    m_per, k = x.shape
    _, n_per = w_mat.shape

    x8 = x.astype(jnp.float8_e4m3fn)
    wb = w_mat.astype(jnp.bfloat16)
    sx = scale_x.reshape(1, 1).astype(jnp.float32)
    sw = scale_w.reshape(1, 1).astype(jnp.float32)

    def body(x_ref, w_ref, sx_ref, sw_ref, out_ref, xg_ref,
             fs_sems, fr_sems, bs_sems, br_sems):
        my = lax.axis_index("i")
        left = lax.rem(my + N_DEV - 1, N_DEV)
        right = lax.rem(my + 1, N_DEV)

        barrier_sem = pltpu.get_barrier_semaphore()
        for nbr in (left, right):
            pl.semaphore_signal(
                barrier_sem, inc=1,
                device_id=(nbr,), device_id_type=pl.DeviceIdType.MESH,
            )
        pl.semaphore_wait(barrier_sem, 2)

        alpha = sx_ref[0, 0] * sw_ref[0, 0]

        def gemm(origin):
            chunk = xg_ref[origin].astype(jnp.bfloat16)
            acc = jnp.dot(chunk, w_ref[:, :], preferred_element_type=jnp.float32)
            out_ref[pl.ds(origin * m_per, m_per), :] = jnp.maximum(acc * alpha, 0.0)

        xg_ref[my] = x_ref[:, :]

        def mk_fwd(h):
            origin = lax.rem(my - h + N_DEV, N_DEV)
            return pltpu.make_async_remote_copy(
                src_ref=xg_ref.at[origin],
                dst_ref=xg_ref.at[origin],
                send_sem=fs_sems.at[h],
                recv_sem=fr_sems.at[h],
                device_id=(right,),
                device_id_type=pl.DeviceIdType.MESH,
            )

        def mk_bwd(h):
            origin = lax.rem(my + h, N_DEV)
            return pltpu.make_async_remote_copy(
                src_ref=xg_ref.at[origin],
                dst_ref=xg_ref.at[origin],
                send_sem=bs_sems.at[h],
                recv_sem=br_sems.at[h],
                device_id=(left,),
                device_id_type=pl.DeviceIdType.MESH,
            )

        descs = []
        f = mk_fwd(0)
        f.start()
        b = mk_bwd(0)
        b.start()
        descs += [f, b]
        gemm(my)
        for h in range(1, FWD):
            f.wait_recv()
            f = mk_fwd(h)
            f.start()
            descs.append(f)
            b.wait_recv()
            if h < BWD:
                b = mk_bwd(h)
                b.start()
                descs.append(b)
            gemm(lax.rem(my - h + N_DEV, N_DEV))
            gemm(lax.rem(my + h, N_DEV))
        f.wait_recv()
        gemm(lax.rem(my + FWD, N_DEV))
        for d in descs:
            d.wait_send()

    out_shape = jax.ShapeDtypeStruct((N_DEV * m_per, n_per), jnp.float32)
    return pl.pallas_call(
        body,
        out_shape=out_shape,
        in_specs=[
            pl.BlockSpec(memory_space=pltpu.VMEM),
            pl.BlockSpec(memory_space=pltpu.VMEM),
            pl.BlockSpec(memory_space=pltpu.SMEM),
            pl.BlockSpec(memory_space=pltpu.SMEM),
        ],
        out_specs=pl.BlockSpec(memory_space=pltpu.VMEM),
        scratch_shapes=[
            pltpu.VMEM((N_DEV, m_per, k), jnp.float8_e4m3fn),
            pltpu.SemaphoreType.DMA((FWD,)),
            pltpu.SemaphoreType.DMA((FWD,)),
            pltpu.SemaphoreType.DMA((BWD,)),
            pltpu.SemaphoreType.DMA((BWD,)),
        ],
        compiler_params=pltpu.CompilerParams(collective_id=0),
    )(x8, wb, sx, sw)


# device time: 187950 ns/iter; 1.2908x vs baseline; 1.0242x over previous
import jax
import jax.numpy as jnp
from jax import lax
from jax.experimental import pallas as pl
from jax.experimental.pallas import tpu as pltpu

N_DEV = 32
FWD = N_DEV // 2
BWD = N_DEV - 1 - FWD


def kernel(x, w_mat, scale_x, scale_w):
    m_per, k = x.shape
    _, n_per = w_mat.shape
    half_m = m_per // 2

    sx = scale_x.reshape(1, 1).astype(jnp.float32)
    sw = scale_w.reshape(1, 1).astype(jnp.float32)

    def body(x_ref, w_ref, sx_ref, sw_ref, out_ref, xg_ref, wb_ref,
             fs_sems, fr_sems, bs_sems, br_sems):
        my = lax.axis_index("i")
        left = lax.rem(my + N_DEV - 1, N_DEV)
        right = lax.rem(my + 1, N_DEV)

        barrier_sem = pltpu.get_barrier_semaphore()
        for nbr in (left, right):
            pl.semaphore_signal(
                barrier_sem, inc=1,
                device_id=(nbr,), device_id_type=pl.DeviceIdType.MESH,
            )
        pl.semaphore_wait(barrier_sem, 2)

        alpha = sx_ref[0, 0] * sw_ref[0, 0]

        def gemm(origin):
            chunk = xg_ref[origin].astype(jnp.bfloat16)
            acc = jnp.dot(chunk, wb_ref[:, :], preferred_element_type=jnp.float32)
            out_ref[pl.ds(origin * m_per, m_per), :] = jnp.maximum(acc * alpha, 0.0)

        xg_ref[my] = x_ref[:, :].astype(jnp.float8_e4m3fn)

        def mk(h, half, fwd):
            if fwd:
                origin = lax.rem(my - h + N_DEV, N_DEV)
                ss, rs, tgt = fs_sems, fr_sems, right
            else:
                origin = lax.rem(my + h, N_DEV)
                ss, rs, tgt = bs_sems, br_sems, left
            sl = xg_ref.at[origin, pl.ds(half * half_m, half_m)]
            return pltpu.make_async_remote_copy(
                src_ref=sl, dst_ref=sl,
                send_sem=ss.at[2 * h + half],
                recv_sem=rs.at[2 * h + half],
                device_id=(tgt,),
                device_id_type=pl.DeviceIdType.MESH,
            )

        fd = {}
        bd = {}
        for half in (0, 1):
            fd[half] = mk(0, half, True)
            fd[half].start()
            bd[half] = mk(0, half, False)
            bd[half].start()
        wb_ref[:, :] = w_ref[:, :].astype(jnp.bfloat16)
        gemm(my)
        for h in range(1, FWD):
            for half in (0, 1):
                fd[half].wait_recv()
                fd[half].wait_send()
                nd = mk(h, half, True)
                nd.start()
                fd[half] = nd
            for half in (0, 1):
                bd[half].wait_recv()
                bd[half].wait_send()
                if h < BWD:
                    nd = mk(h, half, False)
                    nd.start()
                    bd[half] = nd
            gemm(lax.rem(my - h + N_DEV, N_DEV))
            gemm(lax.rem(my + h, N_DEV))
        for half in (0, 1):
            fd[half].wait_recv()
            fd[half].wait_send()
        gemm(lax.rem(my + FWD, N_DEV))

    out_shape = jax.ShapeDtypeStruct((N_DEV * m_per, n_per), jnp.float32)
    return pl.pallas_call(
        body,
        out_shape=out_shape,
        in_specs=[
            pl.BlockSpec(memory_space=pltpu.VMEM),
            pl.BlockSpec(memory_space=pltpu.VMEM),
            pl.BlockSpec(memory_space=pltpu.SMEM),
            pl.BlockSpec(memory_space=pltpu.SMEM),
        ],
        out_specs=pl.BlockSpec(memory_space=pltpu.VMEM),
        scratch_shapes=[
            pltpu.VMEM((N_DEV, m_per, k), jnp.float8_e4m3fn),
            pltpu.VMEM((k, n_per), jnp.bfloat16),
            pltpu.SemaphoreType.DMA((2 * FWD,)),
            pltpu.SemaphoreType.DMA((2 * FWD,)),
            pltpu.SemaphoreType.DMA((2 * BWD,)),
            pltpu.SemaphoreType.DMA((2 * BWD,)),
        ],
        compiler_params=pltpu.CompilerParams(collective_id=0),
    )(x, w_mat, sx, sw)
